# baseline (speedup 1.0000x reference)
import functools

import jax
import jax.numpy as jnp
from jax import lax
from jax.experimental import pallas as pl
from jax.experimental.pallas import tpu as pltpu
from jax.experimental.pallas import tpu_sc as plsc

_NC = 2
_NS = 16
_NW = _NC * _NS
_CH = 80
_SEG = 128
_K = 2


def _tc_matmul(x, w_gcn, w_ab, bias_ab):
    n, d = x.shape
    dh = d // 2
    blk = 400

    def body(x_ref, wg_ref, wab_ref, bab_ref, xl_ref, ab_ref):
        res = jnp.dot(x_ref[...], wg_ref[...], preferred_element_type=jnp.float32)
        xl_ref[0] = res[:, :dh]
        xl_ref[1] = res[:, dh:]
        ab_ref[...] = (
            jnp.dot(x_ref[...], wab_ref[...], preferred_element_type=jnp.float32)
            + bab_ref[...]
        )

    return pl.pallas_call(
        body,
        grid=(n // blk,),
        in_specs=[
            pl.BlockSpec((blk, d), lambda i: (i, 0)),
            pl.BlockSpec((d, d), lambda i: (0, 0)),
            pl.BlockSpec((d, 8), lambda i: (0, 0)),
            pl.BlockSpec((1, 8), lambda i: (0, 0)),
        ],
        out_specs=[
            pl.BlockSpec((2, blk, dh), lambda i: (0, i, 0)),
            pl.BlockSpec((blk, 8), lambda i: (i, 0)),
        ],
        out_shape=[
            jax.ShapeDtypeStruct((2, n, dh), jnp.float32),
            jax.ShapeDtypeStruct((n, 8), jnp.float32),
        ],
    )(x, w_gcn, w_ab, bias_ab)


def _sc_edge_weights(src3, dst3, a, b):
    _, rpw, ch = src3.shape
    n = a.shape[0]
    seg = 2000
    mesh = plsc.VectorSubcoreMesh(core_axis_name="c", subcore_axis_name="s")

    @functools.partial(
        pl.kernel,
        out_type=(
            jax.ShapeDtypeStruct((_NW, rpw, ch), jnp.float32),
            jax.ShapeDtypeStruct((_NC * n,), jnp.float32),
        ),
        mesh=mesh,
        scratch_types=(
            pltpu.VMEM((n,), jnp.float32),
            pltpu.VMEM((n,), jnp.float32),
            pltpu.VMEM((rpw, ch), jnp.int32),
            pltpu.VMEM((rpw, ch), jnp.int32),
            pltpu.VMEM((rpw, ch), jnp.float32),
            pltpu.VMEM((seg,), jnp.float32),
            pltpu.VMEM_SHARED((n,), jnp.float32),
            pltpu.SemaphoreType.DMA,
        ),
        compiler_params=pltpu.CompilerParams(needs_layout_passes=False),
    )
    def kern(src_hbm, dst_hbm, a_hbm, b_hbm, ew_hbm, degp_hbm,
             a_v, b_v, src_v, dst_v, ew_v, stage_v, deg_sh, sem):
        cid = lax.axis_index("c")
        sid = lax.axis_index("s")
        wid = cid * _NS + sid

        @pl.when(sid < n // seg)
        def _():
            for t in range(seg // 16):
                stage_v[pl.ds(t * 16, 16)] = jnp.zeros((16,), jnp.float32)
            pltpu.sync_copy(stage_v, deg_sh.at[pl.ds(sid * seg, seg)])

        pltpu.sync_copy(a_hbm, a_v)
        pltpu.sync_copy(b_hbm, b_v)
        pltpu.sync_copy(src_hbm.at[wid], src_v)
        pltpu.sync_copy(dst_hbm.at[wid], dst_v)
        plsc.subcore_barrier()

        def chunk(r_):
            for g in range(ch // 16):
                sv = src_v[r_, pl.ds(g * 16, 16)]
                dv = dst_v[r_, pl.ds(g * 16, 16)]
                av = plsc.load_gather(a_v, [sv])
                bv = plsc.load_gather(b_v, [dv])
                ew = 1.0 / (1.0 + jnp.exp(-(av + bv)))
                ew_v[r_, pl.ds(g * 16, 16)] = ew

        pl.loop(0, rpw)(chunk)

        def fire(r0):
            descs = [
                pltpu.async_copy(
                    ew_v.at[r0 + j], deg_sh.at[dst_v.at[r0 + j]], sem, add=True
                )
                for j in range(25)
            ]
            for de in descs:
                de.wait()

        pl.loop(0, rpw, step=25)(fire)

        pltpu.sync_copy(ew_v, ew_hbm.at[wid])
        plsc.subcore_barrier()

        @pl.when(sid < n // seg)
        def _():
            pltpu.sync_copy(deg_sh.at[pl.ds(sid * seg, seg)], stage_v)
            pltpu.sync_copy(
                stage_v, degp_hbm.at[pl.ds(cid * n + sid * seg, seg)]
            )

    return kern(src3, dst3, a, b)


def _newton_rsqrt(v):
    i = plsc.bitcast(v, jnp.int32)
    i = 0x5F3759DF - lax.shift_right_logical(i, 1)
    g = plsc.bitcast(i, jnp.float32)
    for _ in range(3):
        g = g * (1.5 - 0.5 * v * g * g)
    return g


def _sc_scatter(src3, dst3, ew3, xl3, degp, bg):
    _, rpw, ch = src3.shape
    _, n, dh = xl3.shape
    nseg = n // _SEG
    rem = n - nseg * _SEG
    rem_tile = nseg % _NS
    mesh = plsc.VectorSubcoreMesh(core_axis_name="c", subcore_axis_name="s")

    @functools.partial(
        pl.kernel,
        out_type=(
            jax.ShapeDtypeStruct((_NC, n, dh), jnp.float32),
            jax.ShapeDtypeStruct((_NC * n, dh), jnp.float32),
        ),
        mesh=mesh,
        scratch_types=(
            pltpu.VMEM((rpw, ch), jnp.int32),
            pltpu.VMEM((rpw, ch), jnp.int32),
            pltpu.VMEM((rpw, ch), jnp.float32),
            pltpu.VMEM((2 * _K, ch, dh), jnp.float32),
            pltpu.VMEM((_SEG, dh), jnp.float32),
            pltpu.VMEM((2, _SEG), jnp.float32),
            pltpu.VMEM((8 * _SEG, ), jnp.float32),
            pltpu.VMEM((dh * 2,), jnp.float32),
            pltpu.VMEM_SHARED((n, dh), jnp.float32),
            pltpu.SemaphoreType.DMA,
            pltpu.SemaphoreType.DMA,
        ),
        compiler_params=pltpu.CompilerParams(
            needs_layout_passes=False, use_tc_tiling_on_sc=False
        ),
    )
    def kern(src_hbm, dst_hbm, ew_hbm, xl_hbm, degp_hbm, bg_hbm,
             out_hbm, y_hbm,
             src_v, dst_v, ew_v, rows_v, stage_v, p_v, dis_v, bg_v,
             acc_sh, gsem, ssem):
        cid = lax.axis_index("c")
        sid = lax.axis_index("s")

        pltpu.sync_copy(src_hbm.at[sid], src_v)
        pltpu.sync_copy(dst_hbm.at[sid], dst_v)
        pltpu.sync_copy(ew_hbm.at[sid], ew_v)
        pltpu.sync_copy(bg_hbm, bg_v)

        def remap(r_):
            for g in range(ch // 16):
                sv = src_v[r_, pl.ds(g * 16, 16)]
                src_v[r_, pl.ds(g * 16, 16)] = sv + cid * n

        pl.loop(0, rpw)(remap)

        def norm_chunk(k, c, sz):
            base = k * _SEG
            pltpu.sync_copy(
                xl_hbm.at[cid, pl.ds(base, sz)], stage_v.at[pl.ds(0, sz)]
            )
            pltpu.sync_copy(degp_hbm.at[pl.ds(base, sz)], p_v.at[0, pl.ds(0, sz)])
            pltpu.sync_copy(
                degp_hbm.at[pl.ds(n + base, sz)], p_v.at[1, pl.ds(0, sz)]
            )

            def grp(g):
                deg = 1.0 + p_v[0, pl.ds(g * 16, 16)] + p_v[1, pl.ds(g * 16, 16)]
                dis = _newton_rsqrt(deg)
                dis_v[pl.ds(c * _SEG + g * 16, 16)] = dis
                for l in range(16):
                    sv = jnp.full((16,), dis[l], jnp.float32)
                    for j in range(dh // 16):
                        stage_v[g * 16 + l, pl.ds(j * 16, 16)] = (
                            stage_v[g * 16 + l, pl.ds(j * 16, 16)] * sv
                        )

            pl.loop(0, sz // 16)(grp)
            pltpu.sync_copy(
                stage_v.at[pl.ds(0, sz)], y_hbm.at[pl.ds(cid * n + base, sz)]
            )
            pltpu.sync_copy(
                stage_v.at[pl.ds(0, sz)], acc_sh.at[pl.ds(base, sz)]
            )

        def pro(k, c):
            norm_chunk(k, c, _SEG)
            return c + 1

        pl.loop(sid, nseg, step=_NS, init_carry=0)(pro)
        if rem:
            @pl.when(sid == rem_tile)
            def _():
                norm_chunk(nseg, nseg // _NS, rem)

        plsc.subcore_barrier()

        def fire_gather(rr, buf):
            @pl.when(rr < rpw)
            def _():
                pltpu.async_copy(y_hbm.at[src_v.at[rr]], rows_v.at[buf], gsem)

        def wait_gather(rr, buf):
            pltpu.make_async_copy(
                y_hbm.at[src_v.at[rr]], rows_v.at[buf], gsem
            ).wait()

        def fire_scatter(rr, buf):
            pltpu.async_copy(
                rows_v.at[buf], acc_sh.at[dst_v.at[rr]], ssem, add=True
            )

        def wait_scatter(rr, buf):
            pltpu.make_async_copy(
                rows_v.at[buf], acc_sh.at[dst_v.at[rr]], ssem
            ).wait()

        def scale(rr, buf):
            def sgroup(g):
                ewg = ew_v[rr, pl.ds(g * 16, 16)]
                for l in range(16):
                    sv = jnp.full((16,), ewg[l], jnp.float32)
                    e = g * 16 + l
                    for j in range(dh // 16):
                        rows_v[buf, e, pl.ds(j * 16, 16)] = (
                            rows_v[buf, e, pl.ds(j * 16, 16)] * sv
                        )

            pl.loop(0, ch // 16)(sgroup)

        def superchunk(s_, set_, drain, prefetch):
            base = set_ * _K
            for k in range(_K):
                wait_gather(s_ * _K + k, base + k)
            for k in range(_K):
                scale(s_ * _K + k, base + k)
            obase = _K - base
            if drain:
                for k in range(_K):
                    wait_scatter((s_ - 1) * _K + k, obase + k)
            for k in range(_K):
                fire_scatter(s_ * _K + k, base + k)
            if prefetch:
                for k in range(_K):
                    fire_gather((s_ + 1) * _K + k, obase + k)

        nss = rpw // _K
        for k in range(_K):
            fire_gather(k, k)
        superchunk(0, 0, drain=False, prefetch=True)

        def ssloop(s_):
            superchunk(s_, 1, drain=True, prefetch=True)
            superchunk(s_ + 1, 0, drain=True, prefetch=True)

        pl.loop(1, nss - 1, step=2)(ssloop)
        if nss % 2 == 0:
            superchunk(nss - 1, 1, drain=True, prefetch=False)
        base_last = ((nss - 1) % 2) * _K
        for k in range(_K):
            wait_scatter((nss - 1) * _K + k, base_last + k)

        plsc.subcore_barrier()

        bj = [
            bg_v[pl.ds(cid * dh + j * 16, 16)] for j in range(dh // 16)
        ]

        def out_chunk(k, c, sz):
            base = k * _SEG
            pltpu.sync_copy(
                acc_sh.at[pl.ds(base, sz)], stage_v.at[pl.ds(0, sz)]
            )

            def grp(g):
                dis = dis_v[pl.ds(c * _SEG + g * 16, 16)]
                for l in range(16):
                    sv = jnp.full((16,), dis[l], jnp.float32)
                    for j in range(dh // 16):
                        stage_v[g * 16 + l, pl.ds(j * 16, 16)] = (
                            stage_v[g * 16 + l, pl.ds(j * 16, 16)] * sv + bj[j]
                        )

            pl.loop(0, sz // 16)(grp)
            pltpu.sync_copy(
                stage_v.at[pl.ds(0, sz)], out_hbm.at[cid, pl.ds(base, sz)]
            )

        def epi(k, c):
            out_chunk(k, c, _SEG)
            return c + 1

        pl.loop(sid, nseg, step=_NS, init_carry=0)(epi)
        if rem:
            @pl.when(sid == rem_tile)
            def _():
                out_chunk(nseg, nseg // _NS, rem)

    return kern(src3, dst3, ew3, xl3, degp, bg)


def kernel(x, edge_index, W_pred, b_pred, W_gcn, b_gcn):
    n, d = x.shape
    e = edge_index.shape[1]
    rpw = e // (_NW * _CH)
    rpw2 = e // (_NS * _CH)

    src = edge_index[0].astype(jnp.int32)
    dst = edge_index[1].astype(jnp.int32)
    src3 = src.reshape(_NW, rpw, _CH)
    dst3 = dst.reshape(_NW, rpw, _CH)

    w_ab = jnp.concatenate(
        [W_pred[:d], W_pred[d:], jnp.zeros((d, 6), jnp.float32)], axis=1
    )
    bias_ab = jnp.zeros((1, 8), jnp.float32).at[0, 1].set(b_pred[0])

    xl3, ab = _tc_matmul(x, W_gcn, w_ab, bias_ab)
    a = ab[:, 0]
    b = ab[:, 1]

    ew3, degp = _sc_edge_weights(src3, dst3, a, b)

    out3, _ = _sc_scatter(
        src.reshape(_NS, rpw2, _CH),
        dst.reshape(_NS, rpw2, _CH),
        ew3.reshape(_NS, rpw2, _CH),
        xl3,
        degp,
        b_gcn,
    )
    return out3.transpose(1, 0, 2).reshape(n, d)


# speedup vs baseline: 1.1205x; 1.1205x over previous
import functools

import jax
import jax.numpy as jnp
from jax import lax
from jax.experimental import pallas as pl
from jax.experimental.pallas import tpu as pltpu
from jax.experimental.pallas import tpu_sc as plsc

_NC = 2
_NS = 16
_NW = _NC * _NS
_CH = 80
_SEG = 128
_K = 1


def _tc_matmul(x, w_gcn, w_ab, bias_ab):
    n, d = x.shape
    dh = d // 2
    blk = 400

    def body(x_ref, wg_ref, wab_ref, bab_ref, xl_ref, ab_ref):
        res = jnp.dot(x_ref[...], wg_ref[...], preferred_element_type=jnp.float32)
        xl_ref[0] = res[:, :dh]
        xl_ref[1] = res[:, dh:]
        ab_ref[...] = (
            jnp.dot(x_ref[...], wab_ref[...], preferred_element_type=jnp.float32)
            + bab_ref[...]
        )

    return pl.pallas_call(
        body,
        grid=(n // blk,),
        in_specs=[
            pl.BlockSpec((blk, d), lambda i: (i, 0)),
            pl.BlockSpec((d, d), lambda i: (0, 0)),
            pl.BlockSpec((d, 8), lambda i: (0, 0)),
            pl.BlockSpec((1, 8), lambda i: (0, 0)),
        ],
        out_specs=[
            pl.BlockSpec((2, blk, dh), lambda i: (0, i, 0)),
            pl.BlockSpec((blk, 8), lambda i: (i, 0)),
        ],
        out_shape=[
            jax.ShapeDtypeStruct((2, n, dh), jnp.float32),
            jax.ShapeDtypeStruct((n, 8), jnp.float32),
        ],
    )(x, w_gcn, w_ab, bias_ab)


def _sc_edge_weights(src3, dst3, a, b):
    _, rpw, ch = src3.shape
    n = a.shape[0]
    seg = 2000
    mesh = plsc.VectorSubcoreMesh(core_axis_name="c", subcore_axis_name="s")

    @functools.partial(
        pl.kernel,
        out_type=(
            jax.ShapeDtypeStruct((_NW, rpw, ch), jnp.float32),
            jax.ShapeDtypeStruct((_NC * n,), jnp.float32),
        ),
        mesh=mesh,
        scratch_types=(
            pltpu.VMEM((n,), jnp.float32),
            pltpu.VMEM((n,), jnp.float32),
            pltpu.VMEM((rpw, ch), jnp.int32),
            pltpu.VMEM((rpw, ch), jnp.int32),
            pltpu.VMEM((rpw, ch), jnp.float32),
            pltpu.VMEM((seg,), jnp.float32),
            pltpu.VMEM_SHARED((n,), jnp.float32),
            pltpu.SemaphoreType.DMA,
        ),
        compiler_params=pltpu.CompilerParams(needs_layout_passes=False),
    )
    def kern(src_hbm, dst_hbm, a_hbm, b_hbm, ew_hbm, degp_hbm,
             a_v, b_v, src_v, dst_v, ew_v, stage_v, deg_sh, sem):
        cid = lax.axis_index("c")
        sid = lax.axis_index("s")
        wid = cid * _NS + sid

        @pl.when(sid < n // seg)
        def _():
            for t in range(seg // 16):
                stage_v[pl.ds(t * 16, 16)] = jnp.zeros((16,), jnp.float32)
            pltpu.sync_copy(stage_v, deg_sh.at[pl.ds(sid * seg, seg)])

        pltpu.sync_copy(a_hbm, a_v)
        pltpu.sync_copy(b_hbm, b_v)
        pltpu.sync_copy(src_hbm.at[wid], src_v)
        pltpu.sync_copy(dst_hbm.at[wid], dst_v)
        plsc.subcore_barrier()

        def chunk(r_):
            for g in range(ch // 16):
                sv = src_v[r_, pl.ds(g * 16, 16)]
                dv = dst_v[r_, pl.ds(g * 16, 16)]
                av = plsc.load_gather(a_v, [sv])
                bv = plsc.load_gather(b_v, [dv])
                ew = 1.0 / (1.0 + jnp.exp(-(av + bv)))
                ew_v[r_, pl.ds(g * 16, 16)] = ew

        pl.loop(0, rpw)(chunk)

        def fire(r0):
            descs = [
                pltpu.async_copy(
                    ew_v.at[r0 + j], deg_sh.at[dst_v.at[r0 + j]], sem, add=True
                )
                for j in range(25)
            ]
            for de in descs:
                de.wait()

        pl.loop(0, rpw, step=25)(fire)

        pltpu.sync_copy(ew_v, ew_hbm.at[wid])
        plsc.subcore_barrier()

        @pl.when(sid < n // seg)
        def _():
            pltpu.sync_copy(deg_sh.at[pl.ds(sid * seg, seg)], stage_v)
            pltpu.sync_copy(
                stage_v, degp_hbm.at[pl.ds(cid * n + sid * seg, seg)]
            )

    return kern(src3, dst3, a, b)


def _newton_rsqrt(v):
    i = plsc.bitcast(v, jnp.int32)
    i = 0x5F3759DF - lax.shift_right_logical(i, 1)
    g = plsc.bitcast(i, jnp.float32)
    for _ in range(3):
        g = g * (1.5 - 0.5 * v * g * g)
    return g


def _sc_scatter(src3, dst3, ew3, xl3, degp, bg):
    _, rpw, ch = src3.shape
    _, n, dh = xl3.shape
    nseg = n // _SEG
    rem = n - nseg * _SEG
    rem_tile = nseg % _NS
    mesh = plsc.VectorSubcoreMesh(core_axis_name="c", subcore_axis_name="s")

    @functools.partial(
        pl.kernel,
        out_type=(
            jax.ShapeDtypeStruct((_NC, n, dh), jnp.float32),
            jax.ShapeDtypeStruct((_NC * n, dh), jnp.float32),
        ),
        mesh=mesh,
        scratch_types=(
            pltpu.VMEM((rpw, ch), jnp.int32),
            pltpu.VMEM((rpw, ch), jnp.int32),
            pltpu.VMEM((rpw, ch), jnp.float32),
            pltpu.VMEM((2 * _K, ch, dh), jnp.float32),
            pltpu.VMEM((_SEG, dh), jnp.float32),
            pltpu.VMEM((2, _SEG), jnp.float32),
            pltpu.VMEM((8 * _SEG, ), jnp.float32),
            pltpu.VMEM((dh * 2,), jnp.float32),
            pltpu.VMEM_SHARED((n, dh), jnp.float32),
            pltpu.SemaphoreType.DMA,
            pltpu.SemaphoreType.DMA,
        ),
        compiler_params=pltpu.CompilerParams(
            needs_layout_passes=False, use_tc_tiling_on_sc=False
        ),
    )
    def kern(src_hbm, dst_hbm, ew_hbm, xl_hbm, degp_hbm, bg_hbm,
             out_hbm, y_hbm,
             src_v, dst_v, ew_v, rows_v, stage_v, p_v, dis_v, bg_v,
             acc_sh, gsem, ssem):
        cid = lax.axis_index("c")
        sid = lax.axis_index("s")

        pltpu.sync_copy(src_hbm.at[sid], src_v)
        pltpu.sync_copy(dst_hbm.at[sid], dst_v)
        pltpu.sync_copy(ew_hbm.at[sid], ew_v)
        pltpu.sync_copy(bg_hbm, bg_v)

        def remap(r_):
            for g in range(ch // 16):
                sv = src_v[r_, pl.ds(g * 16, 16)]
                src_v[r_, pl.ds(g * 16, 16)] = sv + cid * n

        pl.loop(0, rpw)(remap)

        def norm_chunk(k, c, sz):
            base = k * _SEG
            pltpu.sync_copy(
                xl_hbm.at[cid, pl.ds(base, sz)], stage_v.at[pl.ds(0, sz)]
            )
            pltpu.sync_copy(degp_hbm.at[pl.ds(base, sz)], p_v.at[0, pl.ds(0, sz)])
            pltpu.sync_copy(
                degp_hbm.at[pl.ds(n + base, sz)], p_v.at[1, pl.ds(0, sz)]
            )

            def grp(g):
                deg = 1.0 + p_v[0, pl.ds(g * 16, 16)] + p_v[1, pl.ds(g * 16, 16)]
                dis = _newton_rsqrt(deg)
                dis_v[pl.ds(c * _SEG + g * 16, 16)] = dis
                for l in range(16):
                    sv = jnp.full((16,), dis[l], jnp.float32)
                    for j in range(dh // 16):
                        stage_v[g * 16 + l, pl.ds(j * 16, 16)] = (
                            stage_v[g * 16 + l, pl.ds(j * 16, 16)] * sv
                        )

            pl.loop(0, sz // 16)(grp)
            pltpu.sync_copy(
                stage_v.at[pl.ds(0, sz)], y_hbm.at[pl.ds(cid * n + base, sz)]
            )
            pltpu.sync_copy(
                stage_v.at[pl.ds(0, sz)], acc_sh.at[pl.ds(base, sz)]
            )

        def pro(k, c):
            norm_chunk(k, c, _SEG)
            return c + 1

        pl.loop(sid, nseg, step=_NS, init_carry=0)(pro)
        if rem:
            @pl.when(sid == rem_tile)
            def _():
                norm_chunk(nseg, nseg // _NS, rem)

        plsc.subcore_barrier()

        def fire_gather(rr, buf):
            @pl.when(rr < rpw)
            def _():
                pltpu.async_copy(y_hbm.at[src_v.at[rr]], rows_v.at[buf], gsem)

        def wait_gather(rr, buf):
            pltpu.make_async_copy(
                y_hbm.at[src_v.at[rr]], rows_v.at[buf], gsem
            ).wait()

        def scale(rr, buf):
            def sgroup(g):
                ewg = ew_v[rr, pl.ds(g * 16, 16)]
                for l in range(16):
                    sv = jnp.full((16,), ewg[l], jnp.float32)
                    e = g * 16 + l
                    for j in range(dh // 16):
                        rows_v[buf, e, pl.ds(j * 16, 16)] = (
                            rows_v[buf, e, pl.ds(j * 16, 16)] * sv
                        )

            pl.loop(0, ch // 16)(sgroup)

        fire_gather(0, 0)

        def step2(r_):
            for p_ in range(2):
                rr = r_ + p_
                wait_gather(rr, p_)
                fire_gather(rr + 1, 1 - p_)
                scale(rr, p_)
                pltpu.sync_copy(
                    rows_v.at[p_], acc_sh.at[dst_v.at[rr]], add=True
                )

        pl.loop(0, rpw, step=2)(step2)

        plsc.subcore_barrier()

        bj = [
            bg_v[pl.ds(cid * dh + j * 16, 16)] for j in range(dh // 16)
        ]

        def out_chunk(k, c, sz):
            base = k * _SEG
            pltpu.sync_copy(
                acc_sh.at[pl.ds(base, sz)], stage_v.at[pl.ds(0, sz)]
            )

            def grp(g):
                dis = dis_v[pl.ds(c * _SEG + g * 16, 16)]
                for l in range(16):
                    sv = jnp.full((16,), dis[l], jnp.float32)
                    for j in range(dh // 16):
                        stage_v[g * 16 + l, pl.ds(j * 16, 16)] = (
                            stage_v[g * 16 + l, pl.ds(j * 16, 16)] * sv + bj[j]
                        )

            pl.loop(0, sz // 16)(grp)
            pltpu.sync_copy(
                stage_v.at[pl.ds(0, sz)], out_hbm.at[cid, pl.ds(base, sz)]
            )

        def epi(k, c):
            out_chunk(k, c, _SEG)
            return c + 1

        pl.loop(sid, nseg, step=_NS, init_carry=0)(epi)
        if rem:
            @pl.when(sid == rem_tile)
            def _():
                out_chunk(nseg, nseg // _NS, rem)

    return kern(src3, dst3, ew3, xl3, degp, bg)


def kernel(x, edge_index, W_pred, b_pred, W_gcn, b_gcn):
    n, d = x.shape
    e = edge_index.shape[1]
    rpw = e // (_NW * _CH)
    rpw2 = e // (_NS * _CH)

    src = edge_index[0].astype(jnp.int32)
    dst = edge_index[1].astype(jnp.int32)
    src3 = src.reshape(_NW, rpw, _CH)
    dst3 = dst.reshape(_NW, rpw, _CH)

    w_ab = jnp.concatenate(
        [W_pred[:d], W_pred[d:], jnp.zeros((d, 6), jnp.float32)], axis=1
    )
    bias_ab = jnp.zeros((1, 8), jnp.float32).at[0, 1].set(b_pred[0])

    xl3, ab = _tc_matmul(x, W_gcn, w_ab, bias_ab)
    a = ab[:, 0]
    b = ab[:, 1]

    ew3, degp = _sc_edge_weights(src3, dst3, a, b)

    out3, _ = _sc_scatter(
        src.reshape(_NS, rpw2, _CH),
        dst.reshape(_NS, rpw2, _CH),
        ew3.reshape(_NS, rpw2, _CH),
        xl3,
        degp,
        b_gcn,
    )
    return out3.transpose(1, 0, 2).reshape(n, d)


# speedup vs baseline: 1.6690x; 1.4895x over previous
import functools

import jax
import jax.numpy as jnp
from jax import lax
from jax.experimental import pallas as pl
from jax.experimental.pallas import tpu as pltpu
from jax.experimental.pallas import tpu_sc as plsc

_NC = 2
_NS = 16
_NW = _NC * _NS
_CH = 80


def _tc_matmul(x, w_ext, bias_row):
    n, d = x.shape
    dw = w_ext.shape[1]
    blk = 400

    def body(x_ref, w_ref, b_ref, o_ref):
        o_ref[...] = (
            jnp.dot(x_ref[...], w_ref[...], preferred_element_type=jnp.float32)
            + b_ref[...]
        )

    return pl.pallas_call(
        body,
        grid=(n // blk,),
        in_specs=[
            pl.BlockSpec((blk, d), lambda i: (i, 0)),
            pl.BlockSpec((d, dw), lambda i: (0, 0)),
            pl.BlockSpec((1, dw), lambda i: (0, 0)),
        ],
        out_specs=pl.BlockSpec((blk, dw), lambda i: (i, 0)),
        out_shape=jax.ShapeDtypeStruct((n, dw), jnp.float32),
    )(x, w_ext, bias_row)


def _row_partition(n_rows, n_tiles, max_chunk):
    per = -(-n_rows // n_tiles)
    per = ((per + 7) // 8) * 8
    parts = []
    base = 0
    for _ in range(n_tiles):
        cnt = max(0, min(per, n_rows - base))
        sizes = []
        left = cnt
        while left > 0:
            sz = min(max_chunk, left)
            sizes.append(sz)
            left -= sz
        parts.append((base, sizes))
        base += cnt
    return parts


def _sc_edge_weights(src3, dst3, a, b):
    _, rpw, ch = src3.shape
    n = a.shape[0]
    seg = 2000
    mesh = plsc.VectorSubcoreMesh(core_axis_name="c", subcore_axis_name="s")

    @functools.partial(
        pl.kernel,
        out_type=(
            jax.ShapeDtypeStruct((_NW, rpw, ch), jnp.float32),
            jax.ShapeDtypeStruct((_NC * n,), jnp.float32),
        ),
        mesh=mesh,
        scratch_types=(
            pltpu.VMEM((n,), jnp.float32),
            pltpu.VMEM((n,), jnp.float32),
            pltpu.VMEM((rpw, ch), jnp.int32),
            pltpu.VMEM((rpw, ch), jnp.int32),
            pltpu.VMEM((rpw, ch), jnp.float32),
            pltpu.VMEM((seg,), jnp.float32),
            pltpu.VMEM_SHARED((n,), jnp.float32),
            pltpu.SemaphoreType.DMA,
        ),
        compiler_params=pltpu.CompilerParams(needs_layout_passes=False),
    )
    def kern(src_hbm, dst_hbm, a_hbm, b_hbm, ew_hbm, degp_hbm,
             a_v, b_v, src_v, dst_v, ew_v, stage_v, deg_sh, sem):
        cid = lax.axis_index("c")
        sid = lax.axis_index("s")
        wid = cid * _NS + sid

        @pl.when(sid < n // seg)
        def _():
            for t in range(seg // 16):
                stage_v[pl.ds(t * 16, 16)] = jnp.zeros((16,), jnp.float32)
            pltpu.sync_copy(stage_v, deg_sh.at[pl.ds(sid * seg, seg)])

        pltpu.sync_copy(a_hbm, a_v)
        pltpu.sync_copy(b_hbm, b_v)
        pltpu.sync_copy(src_hbm.at[wid], src_v)
        pltpu.sync_copy(dst_hbm.at[wid], dst_v)
        plsc.subcore_barrier()

        def chunk(r_):
            for g in range(ch // 16):
                sv = src_v[r_, pl.ds(g * 16, 16)]
                dv = dst_v[r_, pl.ds(g * 16, 16)]
                av = plsc.load_gather(a_v, [sv])
                bv = plsc.load_gather(b_v, [dv])
                ew = 1.0 / (1.0 + jnp.exp(-(av + bv)))
                ew_v[r_, pl.ds(g * 16, 16)] = ew

        pl.loop(0, rpw)(chunk)

        def fire(r0):
            descs = [
                pltpu.async_copy(
                    ew_v.at[r0 + j], deg_sh.at[dst_v.at[r0 + j]], sem, add=True
                )
                for j in range(25)
            ]
            for de in descs:
                de.wait()

        pl.loop(0, rpw, step=25)(fire)

        pltpu.sync_copy(ew_v, ew_hbm.at[wid])
        plsc.subcore_barrier()

        @pl.when(sid < n // seg)
        def _():
            pltpu.sync_copy(deg_sh.at[pl.ds(sid * seg, seg)], stage_v)
            pltpu.sync_copy(
                stage_v, degp_hbm.at[pl.ds(cid * n + sid * seg, seg)]
            )

    return kern(src3, dst3, a, b)


def _sc_scatter(src3, dst3, ew3, y2):
    _, rpw, ch = src3.shape
    n2, dh = y2.shape
    n = n2 // 2
    stage_rows = 128
    parts = _row_partition(n, _NS, stage_rows)
    mesh = plsc.VectorSubcoreMesh(core_axis_name="c", subcore_axis_name="s")

    @functools.partial(
        pl.kernel,
        out_type=jax.ShapeDtypeStruct((_NC, n, dh), jnp.float32),
        mesh=mesh,
        scratch_types=(
            pltpu.VMEM((rpw, ch), jnp.int32),
            pltpu.VMEM((rpw, ch), jnp.int32),
            pltpu.VMEM((rpw, ch), jnp.float32),
            pltpu.VMEM((2, ch, dh), jnp.float32),
            pltpu.VMEM((128, dh), jnp.float32),
            pltpu.VMEM_SHARED((n, dh), jnp.float32),
            pltpu.SemaphoreType.DMA,
        ),
        compiler_params=pltpu.CompilerParams(
            needs_layout_passes=False, use_tc_tiling_on_sc=False
        ),
    )
    def kern(src_hbm, dst_hbm, ew_hbm, y_hbm, msg_hbm,
             src_v, dst_v, ew_v, rows_v, stage_v, acc_sh, gsem):
        cid = lax.axis_index("c")
        sid = lax.axis_index("s")

        def zrow(t):
            for j in range(dh // 16):
                stage_v[t, pl.ds(j * 16, 16)] = jnp.zeros((16,), jnp.float32)

        pl.loop(0, 128)(zrow)
        for t, (base, sizes) in enumerate(parts):
            @pl.when(sid == t)
            def _(base=base, sizes=sizes):
                off = 0
                for sz in sizes:
                    pltpu.sync_copy(
                        stage_v.at[pl.ds(0, sz)],
                        acc_sh.at[pl.ds(base + off, sz)],
                    )
                    off += sz

        pltpu.sync_copy(src_hbm.at[sid], src_v)
        pltpu.sync_copy(dst_hbm.at[sid], dst_v)
        pltpu.sync_copy(ew_hbm.at[sid], ew_v)

        def remap(r_):
            for g in range(ch // 16):
                sv = src_v[r_, pl.ds(g * 16, 16)]
                src_v[r_, pl.ds(g * 16, 16)] = sv * 2 + cid

        pl.loop(0, rpw)(remap)
        plsc.subcore_barrier()

        def process(rr, buf):
            pltpu.make_async_copy(
                y_hbm.at[src_v.at[rr]], rows_v.at[buf], gsem
            ).wait()

            @pl.when(rr + 1 < rpw)
            def _():
                pltpu.async_copy(
                    y_hbm.at[src_v.at[rr + 1]], rows_v.at[1 - buf], gsem
                )

            def sgroup(g):
                ewg = ew_v[rr, pl.ds(g * 16, 16)]
                for l in range(16):
                    sv = jnp.full((16,), ewg[l], jnp.float32)
                    e = g * 16 + l
                    for j in range(dh // 16):
                        rows_v[buf, e, pl.ds(j * 16, 16)] = (
                            rows_v[buf, e, pl.ds(j * 16, 16)] * sv
                        )

            pl.loop(0, ch // 16)(sgroup)

            pltpu.sync_copy(rows_v.at[buf], acc_sh.at[dst_v.at[rr]], add=True)

        pltpu.async_copy(y_hbm.at[src_v.at[0]], rows_v.at[0], gsem)

        def step2(r_):
            process(r_, 0)
            process(r_ + 1, 1)

        pl.loop(0, rpw, step=2)(step2)

        plsc.subcore_barrier()
        for t, (base, sizes) in enumerate(parts):
            @pl.when(sid == t)
            def _(base=base, sizes=sizes):
                off = 0
                for sz in sizes:
                    pltpu.sync_copy(
                        acc_sh.at[pl.ds(base + off, sz)],
                        stage_v.at[pl.ds(0, sz)],
                    )
                    pltpu.sync_copy(
                        stage_v.at[pl.ds(0, sz)],
                        msg_hbm.at[cid, pl.ds(base + off, sz)],
                    )
                    off += sz

    return kern(src3, dst3, ew3, y2)


def _tc_scale(degp_t, x_lin):
    n, d = x_lin.shape
    blk = 400

    def body(p_ref, xl_ref, y_ref):
        deg = 1.0 + p_ref[:, 0:1] + p_ref[:, 1:2]
        y_ref[...] = lax.rsqrt(deg) * xl_ref[...]

    return pl.pallas_call(
        body,
        grid=(n // blk,),
        in_specs=[
            pl.BlockSpec((blk, 2), lambda i: (i, 0)),
            pl.BlockSpec((blk, d), lambda i: (i, 0)),
        ],
        out_specs=pl.BlockSpec((blk, d), lambda i: (i, 0)),
        out_shape=jax.ShapeDtypeStruct((n, d), jnp.float32),
    )(degp_t, x_lin)


def _tc_combine(degp_t, msg, y, bias):
    n, d = y.shape
    blk = 400

    def body(p_ref, m_ref, y_ref, b_ref, o_ref):
        deg = 1.0 + p_ref[:, 0:1] + p_ref[:, 1:2]
        dis = lax.rsqrt(deg)
        m_full = jnp.concatenate([m_ref[0], m_ref[1]], axis=1)
        o_ref[...] = dis * (m_full + y_ref[...]) + b_ref[...]

    return pl.pallas_call(
        body,
        grid=(n // blk,),
        in_specs=[
            pl.BlockSpec((blk, 2), lambda i: (i, 0)),
            pl.BlockSpec((2, blk, d // 2), lambda i: (0, i, 0)),
            pl.BlockSpec((blk, d), lambda i: (i, 0)),
            pl.BlockSpec((1, d), lambda i: (0, 0)),
        ],
        out_specs=pl.BlockSpec((blk, d), lambda i: (i, 0)),
        out_shape=jax.ShapeDtypeStruct((n, d), jnp.float32),
    )(degp_t, msg, y, bias)


def kernel(x, edge_index, W_pred, b_pred, W_gcn, b_gcn):
    n, d = x.shape
    e = edge_index.shape[1]
    rpw = e // (_NW * _CH)
    rpw2 = e // (_NS * _CH)

    src = edge_index[0].astype(jnp.int32)
    dst = edge_index[1].astype(jnp.int32)
    src3 = src.reshape(_NW, rpw, _CH)
    dst3 = dst.reshape(_NW, rpw, _CH)

    w_ext = jnp.concatenate(
        [W_gcn, W_pred[:d], W_pred[d:], jnp.zeros((d, d - 2), jnp.float32)],
        axis=1,
    )
    bias_row = jnp.zeros((1, 2 * d), jnp.float32).at[0, d + 1].set(b_pred[0])

    xe = _tc_matmul(x, w_ext, bias_row)
    x_lin = xe[:, :d]
    a = xe[:, d]
    b = xe[:, d + 1]

    ew3, degp = _sc_edge_weights(src3, dst3, a, b)
    degp_t = degp.reshape(_NC, n).T

    y = _tc_scale(degp_t, x_lin)
    msg = _sc_scatter(
        src.reshape(_NS, rpw2, _CH),
        dst.reshape(_NS, rpw2, _CH),
        ew3.reshape(_NS, rpw2, _CH),
        y.reshape(2 * n, d // 2),
    )
    out = _tc_combine(degp_t, msg, y, b_gcn.reshape(1, d))
    return out


# speedup vs baseline: 1.9069x; 1.1426x over previous
import functools

import jax
import jax.numpy as jnp
from jax import lax
from jax.experimental import pallas as pl
from jax.experimental.pallas import tpu as pltpu
from jax.experimental.pallas import tpu_sc as plsc

_NC = 2
_NS = 16
_NW = _NC * _NS
_CH = 80


def _tc_matmul(x, w_ext, bias_row):
    n, d = x.shape
    dw = w_ext.shape[1]
    blk = 400

    def body(x_ref, w_ref, b_ref, o_ref):
        o_ref[...] = (
            jnp.dot(x_ref[...], w_ref[...], preferred_element_type=jnp.float32)
            + b_ref[...]
        )

    return pl.pallas_call(
        body,
        grid=(n // blk,),
        in_specs=[
            pl.BlockSpec((blk, d), lambda i: (i, 0)),
            pl.BlockSpec((d, dw), lambda i: (0, 0)),
            pl.BlockSpec((1, dw), lambda i: (0, 0)),
        ],
        out_specs=pl.BlockSpec((blk, dw), lambda i: (i, 0)),
        out_shape=jax.ShapeDtypeStruct((n, dw), jnp.float32),
    )(x, w_ext, bias_row)


def _row_partition(n_rows, n_tiles, max_chunk):
    per = -(-n_rows // n_tiles)
    per = ((per + 7) // 8) * 8
    parts = []
    base = 0
    for _ in range(n_tiles):
        cnt = max(0, min(per, n_rows - base))
        sizes = []
        left = cnt
        while left > 0:
            sz = min(max_chunk, left)
            sizes.append(sz)
            left -= sz
        parts.append((base, sizes))
        base += cnt
    return parts


def _sc_edge_weights(src3, dst3, a, b):
    _, rpw, ch = src3.shape
    n = a.shape[0]
    seg = 2000
    mesh = plsc.VectorSubcoreMesh(core_axis_name="c", subcore_axis_name="s")

    @functools.partial(
        pl.kernel,
        out_type=(
            jax.ShapeDtypeStruct((_NW, rpw, ch), jnp.float32),
            jax.ShapeDtypeStruct((_NC * n,), jnp.float32),
        ),
        mesh=mesh,
        scratch_types=(
            pltpu.VMEM((n,), jnp.float32),
            pltpu.VMEM((n,), jnp.float32),
            pltpu.VMEM((rpw, ch), jnp.int32),
            pltpu.VMEM((rpw, ch), jnp.int32),
            pltpu.VMEM((rpw, ch), jnp.float32),
            pltpu.VMEM((seg,), jnp.float32),
            pltpu.VMEM_SHARED((n,), jnp.float32),
            pltpu.SemaphoreType.DMA,
        ),
        compiler_params=pltpu.CompilerParams(needs_layout_passes=False),
    )
    def kern(src_hbm, dst_hbm, a_hbm, b_hbm, ew_hbm, degp_hbm,
             a_v, b_v, src_v, dst_v, ew_v, stage_v, deg_sh, sem):
        cid = lax.axis_index("c")
        sid = lax.axis_index("s")
        wid = cid * _NS + sid

        @pl.when(sid < n // seg)
        def _():
            for t in range(seg // 16):
                stage_v[pl.ds(t * 16, 16)] = jnp.zeros((16,), jnp.float32)
            pltpu.sync_copy(stage_v, deg_sh.at[pl.ds(sid * seg, seg)])

        pltpu.sync_copy(a_hbm, a_v)
        pltpu.sync_copy(b_hbm, b_v)
        pltpu.sync_copy(src_hbm.at[wid], src_v)
        pltpu.sync_copy(dst_hbm.at[wid], dst_v)
        plsc.subcore_barrier()

        def chunk(r_):
            for g in range(ch // 16):
                sv = src_v[r_, pl.ds(g * 16, 16)]
                dv = dst_v[r_, pl.ds(g * 16, 16)]
                av = plsc.load_gather(a_v, [sv])
                bv = plsc.load_gather(b_v, [dv])
                ew = 1.0 / (1.0 + jnp.exp(-(av + bv)))
                ew_v[r_, pl.ds(g * 16, 16)] = ew

        pl.loop(0, rpw)(chunk)

        def fire(r0):
            descs = [
                pltpu.async_copy(
                    ew_v.at[r0 + j], deg_sh.at[dst_v.at[r0 + j]], sem, add=True
                )
                for j in range(25)
            ]
            for de in descs:
                de.wait()

        pl.loop(0, rpw, step=25)(fire)

        pltpu.sync_copy(ew_v, ew_hbm.at[wid])
        plsc.subcore_barrier()

        @pl.when(sid < n // seg)
        def _():
            pltpu.sync_copy(deg_sh.at[pl.ds(sid * seg, seg)], stage_v)
            pltpu.sync_copy(
                stage_v, degp_hbm.at[pl.ds(cid * n + sid * seg, seg)]
            )

    return kern(src3, dst3, a, b)


def _sc_scatter(src3, dst3, ew3, y2):
    _, rpw, ch = src3.shape
    n2, dh = y2.shape
    n = n2 // 2
    stage_rows = 128
    parts = _row_partition(n, _NS, stage_rows)
    mesh = plsc.VectorSubcoreMesh(core_axis_name="c", subcore_axis_name="s")

    @functools.partial(
        pl.kernel,
        out_type=jax.ShapeDtypeStruct((_NC, n, dh), jnp.float32),
        mesh=mesh,
        scratch_types=(
            pltpu.VMEM((rpw, ch), jnp.int32),
            pltpu.VMEM((rpw, ch), jnp.int32),
            pltpu.VMEM((rpw, ch), jnp.float32),
            pltpu.VMEM((3, ch, dh), jnp.float32),
            pltpu.VMEM((128, dh), jnp.float32),
            pltpu.VMEM_SHARED((n, dh), jnp.float32),
            pltpu.SemaphoreType.DMA,
            pltpu.SemaphoreType.DMA,
        ),
        compiler_params=pltpu.CompilerParams(
            needs_layout_passes=False, use_tc_tiling_on_sc=False
        ),
    )
    def kern(src_hbm, dst_hbm, ew_hbm, y_hbm, msg_hbm,
             src_v, dst_v, ew_v, rows_v, stage_v, acc_sh, gsem, ssem):
        cid = lax.axis_index("c")
        sid = lax.axis_index("s")

        def zrow(t):
            for j in range(dh // 16):
                stage_v[t, pl.ds(j * 16, 16)] = jnp.zeros((16,), jnp.float32)

        pl.loop(0, 128)(zrow)
        for t, (base, sizes) in enumerate(parts):
            @pl.when(sid == t)
            def _(base=base, sizes=sizes):
                off = 0
                for sz in sizes:
                    pltpu.sync_copy(
                        stage_v.at[pl.ds(0, sz)],
                        acc_sh.at[pl.ds(base + off, sz)],
                    )
                    off += sz

        pltpu.sync_copy(src_hbm.at[sid], src_v)
        pltpu.sync_copy(dst_hbm.at[sid], dst_v)
        pltpu.sync_copy(ew_hbm.at[sid], ew_v)

        def remap(r_):
            for g in range(ch // 16):
                sv = src_v[r_, pl.ds(g * 16, 16)]
                src_v[r_, pl.ds(g * 16, 16)] = sv * 2 + cid

        pl.loop(0, rpw)(remap)
        plsc.subcore_barrier()

        def fire_gather(rr, buf):
            @pl.when(rr < rpw)
            def _():
                pltpu.async_copy(y_hbm.at[src_v.at[rr]], rows_v.at[buf], gsem)

        def wait_gather(rr, buf):
            pltpu.make_async_copy(
                y_hbm.at[src_v.at[rr]], rows_v.at[buf], gsem
            ).wait()

        def fire_scatter(rr, buf):
            pltpu.async_copy(
                rows_v.at[buf], acc_sh.at[dst_v.at[rr]], ssem, add=True
            )

        def wait_scatter(rr, buf):
            pltpu.make_async_copy(
                rows_v.at[buf], acc_sh.at[dst_v.at[rr]], ssem
            ).wait()

        def scale(rr, buf):
            def sgroup(g):
                ewg = ew_v[rr, pl.ds(g * 16, 16)]
                for l in range(16):
                    sv = jnp.full((16,), ewg[l], jnp.float32)
                    e = g * 16 + l
                    for j in range(dh // 16):
                        rows_v[buf, e, pl.ds(j * 16, 16)] = (
                            rows_v[buf, e, pl.ds(j * 16, 16)] * sv
                        )

            pl.loop(0, ch // 16)(sgroup)

        fire_gather(0, 0)
        fire_gather(1, 1)
        wait_gather(0, 0)
        scale(0, 0)
        fire_scatter(0, 0)
        fire_gather(2, 2)

        def ring(r0):
            for q in range(3):
                rr = r0 + q
                buf = (1 + q) % 3
                wait_gather(rr, buf)
                scale(rr, buf)
                wait_scatter(rr - 1, (buf + 2) % 3)
                fire_scatter(rr, buf)
                fire_gather(rr + 2, (buf + 2) % 3)

        tail0 = 1 + 3 * ((rpw - 4) // 3)
        pl.loop(1, tail0, step=3)(ring)
        for rr in range(tail0, rpw):
            wait_gather(rr, rr % 3)
            scale(rr, rr % 3)
            wait_scatter(rr - 1, (rr - 1) % 3)
            fire_scatter(rr, rr % 3)
            fire_gather(rr + 2, (rr + 2) % 3)
        wait_scatter(rpw - 1, (rpw - 1) % 3)

        plsc.subcore_barrier()
        for t, (base, sizes) in enumerate(parts):
            @pl.when(sid == t)
            def _(base=base, sizes=sizes):
                off = 0
                for sz in sizes:
                    pltpu.sync_copy(
                        acc_sh.at[pl.ds(base + off, sz)],
                        stage_v.at[pl.ds(0, sz)],
                    )
                    pltpu.sync_copy(
                        stage_v.at[pl.ds(0, sz)],
                        msg_hbm.at[cid, pl.ds(base + off, sz)],
                    )
                    off += sz

    return kern(src3, dst3, ew3, y2)


def _tc_scale(degp_t, x_lin):
    n, d = x_lin.shape
    blk = 400

    def body(p_ref, xl_ref, y_ref):
        deg = 1.0 + p_ref[:, 0:1] + p_ref[:, 1:2]
        y_ref[...] = lax.rsqrt(deg) * xl_ref[...]

    return pl.pallas_call(
        body,
        grid=(n // blk,),
        in_specs=[
            pl.BlockSpec((blk, 2), lambda i: (i, 0)),
            pl.BlockSpec((blk, d), lambda i: (i, 0)),
        ],
        out_specs=pl.BlockSpec((blk, d), lambda i: (i, 0)),
        out_shape=jax.ShapeDtypeStruct((n, d), jnp.float32),
    )(degp_t, x_lin)


def _tc_combine(degp_t, msg, y, bias):
    n, d = y.shape
    blk = 400

    def body(p_ref, m_ref, y_ref, b_ref, o_ref):
        deg = 1.0 + p_ref[:, 0:1] + p_ref[:, 1:2]
        dis = lax.rsqrt(deg)
        m_full = jnp.concatenate([m_ref[0], m_ref[1]], axis=1)
        o_ref[...] = dis * (m_full + y_ref[...]) + b_ref[...]

    return pl.pallas_call(
        body,
        grid=(n // blk,),
        in_specs=[
            pl.BlockSpec((blk, 2), lambda i: (i, 0)),
            pl.BlockSpec((2, blk, d // 2), lambda i: (0, i, 0)),
            pl.BlockSpec((blk, d), lambda i: (i, 0)),
            pl.BlockSpec((1, d), lambda i: (0, 0)),
        ],
        out_specs=pl.BlockSpec((blk, d), lambda i: (i, 0)),
        out_shape=jax.ShapeDtypeStruct((n, d), jnp.float32),
    )(degp_t, msg, y, bias)


def kernel(x, edge_index, W_pred, b_pred, W_gcn, b_gcn):
    n, d = x.shape
    e = edge_index.shape[1]
    rpw = e // (_NW * _CH)
    rpw2 = e // (_NS * _CH)

    src = edge_index[0].astype(jnp.int32)
    dst = edge_index[1].astype(jnp.int32)
    src3 = src.reshape(_NW, rpw, _CH)
    dst3 = dst.reshape(_NW, rpw, _CH)

    w_ext = jnp.concatenate(
        [W_gcn, W_pred[:d], W_pred[d:], jnp.zeros((d, d - 2), jnp.float32)],
        axis=1,
    )
    bias_row = jnp.zeros((1, 2 * d), jnp.float32).at[0, d + 1].set(b_pred[0])

    xe = _tc_matmul(x, w_ext, bias_row)
    x_lin = xe[:, :d]
    a = xe[:, d]
    b = xe[:, d + 1]

    ew3, degp = _sc_edge_weights(src3, dst3, a, b)
    degp_t = degp.reshape(_NC, n).T

    y = _tc_scale(degp_t, x_lin)
    msg = _sc_scatter(
        src.reshape(_NS, rpw2, _CH),
        dst.reshape(_NS, rpw2, _CH),
        ew3.reshape(_NS, rpw2, _CH),
        y.reshape(2 * n, d // 2),
    )
    out = _tc_combine(degp_t, msg, y, b_gcn.reshape(1, d))
    return out


# speedup vs baseline: 1.9614x; 1.0285x over previous
import functools

import jax
import jax.numpy as jnp
from jax import lax
from jax.experimental import pallas as pl
from jax.experimental.pallas import tpu as pltpu
from jax.experimental.pallas import tpu_sc as plsc

_NC = 2
_NS = 16
_NW = _NC * _NS
_CH = 80


def _tc_matmul(x, w_ext, bias_row):
    n, d = x.shape
    dw = w_ext.shape[1]
    blk = 400

    def body(x_ref, w_ref, b_ref, o_ref):
        o_ref[...] = (
            jnp.dot(x_ref[...], w_ref[...], preferred_element_type=jnp.float32)
            + b_ref[...]
        )

    return pl.pallas_call(
        body,
        grid=(n // blk,),
        in_specs=[
            pl.BlockSpec((blk, d), lambda i: (i, 0)),
            pl.BlockSpec((d, dw), lambda i: (0, 0)),
            pl.BlockSpec((1, dw), lambda i: (0, 0)),
        ],
        out_specs=pl.BlockSpec((blk, dw), lambda i: (i, 0)),
        out_shape=jax.ShapeDtypeStruct((n, dw), jnp.float32),
    )(x, w_ext, bias_row)


def _row_partition(n_rows, n_tiles, max_chunk):
    per = -(-n_rows // n_tiles)
    per = ((per + 7) // 8) * 8
    parts = []
    base = 0
    for _ in range(n_tiles):
        cnt = max(0, min(per, n_rows - base))
        sizes = []
        left = cnt
        while left > 0:
            sz = min(max_chunk, left)
            sizes.append(sz)
            left -= sz
        parts.append((base, sizes))
        base += cnt
    return parts


def _sc_edge_weights(src3, dst3, a, b):
    _, rpw, ch = src3.shape
    n = a.shape[0]
    seg = 2000
    mesh = plsc.VectorSubcoreMesh(core_axis_name="c", subcore_axis_name="s")

    @functools.partial(
        pl.kernel,
        out_type=(
            jax.ShapeDtypeStruct((_NW, rpw, ch), jnp.float32),
            jax.ShapeDtypeStruct((_NC * n,), jnp.float32),
        ),
        mesh=mesh,
        scratch_types=(
            pltpu.VMEM((n,), jnp.float32),
            pltpu.VMEM((n,), jnp.float32),
            pltpu.VMEM((rpw, ch), jnp.int32),
            pltpu.VMEM((rpw, ch), jnp.int32),
            pltpu.VMEM((rpw, ch), jnp.float32),
            pltpu.VMEM((seg,), jnp.float32),
            pltpu.VMEM_SHARED((n,), jnp.float32),
            pltpu.SemaphoreType.DMA,
        ),
        compiler_params=pltpu.CompilerParams(needs_layout_passes=False),
    )
    def kern(src_hbm, dst_hbm, a_hbm, b_hbm, ew_hbm, degp_hbm,
             a_v, b_v, src_v, dst_v, ew_v, stage_v, deg_sh, sem):
        cid = lax.axis_index("c")
        sid = lax.axis_index("s")
        wid = cid * _NS + sid

        @pl.when(sid < n // seg)
        def _():
            for t in range(seg // 16):
                stage_v[pl.ds(t * 16, 16)] = jnp.zeros((16,), jnp.float32)
            pltpu.sync_copy(stage_v, deg_sh.at[pl.ds(sid * seg, seg)])

        pltpu.sync_copy(a_hbm, a_v)
        pltpu.sync_copy(b_hbm, b_v)
        pltpu.sync_copy(src_hbm.at[wid], src_v)
        pltpu.sync_copy(dst_hbm.at[wid], dst_v)
        plsc.subcore_barrier()

        def chunk(r_):
            for g in range(ch // 16):
                sv = src_v[r_, pl.ds(g * 16, 16)]
                dv = dst_v[r_, pl.ds(g * 16, 16)]
                av = plsc.load_gather(a_v, [sv])
                bv = plsc.load_gather(b_v, [dv])
                ew = 1.0 / (1.0 + jnp.exp(-(av + bv)))
                ew_v[r_, pl.ds(g * 16, 16)] = ew

        pl.loop(0, rpw)(chunk)

        def fire(r0):
            descs = [
                pltpu.async_copy(
                    ew_v.at[r0 + j], deg_sh.at[dst_v.at[r0 + j]], sem, add=True
                )
                for j in range(25)
            ]
            for de in descs:
                de.wait()

        pl.loop(0, rpw, step=25)(fire)

        pltpu.sync_copy(ew_v, ew_hbm.at[wid])
        plsc.subcore_barrier()

        @pl.when(sid < n // seg)
        def _():
            pltpu.sync_copy(deg_sh.at[pl.ds(sid * seg, seg)], stage_v)
            pltpu.sync_copy(
                stage_v, degp_hbm.at[pl.ds(cid * n + sid * seg, seg)]
            )

    return kern(src3, dst3, a, b)


def _newton_rsqrt(v):
    i = plsc.bitcast(v, jnp.int32)
    i = 0x5F3759DF - lax.shift_right_logical(i, 1)
    g = plsc.bitcast(i, jnp.float32)
    for _ in range(3):
        g = g * (1.5 - 0.5 * v * g * g)
    return g


def _sc_scatter(src3, dst3, ew3, xl2, degp):
    _, rpw, ch = src3.shape
    n2, dh = xl2.shape
    n = n2 // 2
    stage_rows = 64
    parts = _row_partition(n, _NS, stage_rows)
    mesh = plsc.VectorSubcoreMesh(core_axis_name="c", subcore_axis_name="s")

    @functools.partial(
        pl.kernel,
        out_type=(
            jax.ShapeDtypeStruct((_NC, n, dh), jnp.float32),
            jax.ShapeDtypeStruct((n,), jnp.float32),
        ),
        mesh=mesh,
        scratch_types=(
            pltpu.VMEM((rpw, ch), jnp.int32),
            pltpu.VMEM((rpw, ch), jnp.int32),
            pltpu.VMEM((rpw, ch), jnp.float32),
            pltpu.VMEM((3, ch, dh), jnp.float32),
            pltpu.VMEM((64, dh), jnp.float32),
            pltpu.VMEM((n,), jnp.float32),
            pltpu.VMEM((2, 640), jnp.float32),
            pltpu.VMEM_SHARED((n, dh), jnp.float32),
            pltpu.SemaphoreType.DMA,
            pltpu.SemaphoreType.DMA,
        ),
        compiler_params=pltpu.CompilerParams(
            needs_layout_passes=False, use_tc_tiling_on_sc=False
        ),
    )
    def kern(src_hbm, dst_hbm, ew_hbm, xl_hbm, degp_hbm, msg_hbm, dis_hbm,
             src_v, dst_v, ew_v, rows_v, stage_v, dis_v, p_v,
             acc_sh, gsem, ssem):
        cid = lax.axis_index("c")
        sid = lax.axis_index("s")

        dbase = sid * 640
        last = n - 640 * (_NS - 1)

        def dgrp(g):
            deg = 1.0 + p_v[0, pl.ds(g * 16, 16)] + p_v[1, pl.ds(g * 16, 16)]
            dis_v[pl.ds(g * 16, 16)] = _newton_rsqrt(deg)

        @pl.when(sid < _NS - 1)
        def _():
            pltpu.sync_copy(degp_hbm.at[pl.ds(dbase, 640)], p_v.at[0])
            pltpu.sync_copy(degp_hbm.at[pl.ds(n + dbase, 640)], p_v.at[1])
            pl.loop(0, 640 // 16)(dgrp)
            pltpu.sync_copy(
                dis_v.at[pl.ds(0, 640)], dis_hbm.at[pl.ds(dbase, 640)]
            )

        @pl.when(sid == _NS - 1)
        def _():
            pltpu.sync_copy(
                degp_hbm.at[pl.ds(dbase, last)], p_v.at[0, pl.ds(0, last)]
            )
            pltpu.sync_copy(
                degp_hbm.at[pl.ds(n + dbase, last)], p_v.at[1, pl.ds(0, last)]
            )
            pl.loop(0, last // 16)(dgrp)
            pltpu.sync_copy(
                dis_v.at[pl.ds(0, last)], dis_hbm.at[pl.ds(dbase, last)]
            )

        def zrow(t):
            for j in range(dh // 16):
                stage_v[t, pl.ds(j * 16, 16)] = jnp.zeros((16,), jnp.float32)

        pl.loop(0, 64)(zrow)
        for t, (base, sizes) in enumerate(parts):
            @pl.when(sid == t)
            def _(base=base, sizes=sizes):
                off = 0
                for sz in sizes:
                    pltpu.sync_copy(
                        stage_v.at[pl.ds(0, sz)],
                        acc_sh.at[pl.ds(base + off, sz)],
                    )
                    off += sz

        pltpu.sync_copy(src_hbm.at[sid], src_v)
        pltpu.sync_copy(dst_hbm.at[sid], dst_v)
        pltpu.sync_copy(ew_hbm.at[sid], ew_v)

        plsc.subcore_barrier()
        pltpu.sync_copy(dis_hbm, dis_v)

        def remap(r_):
            for g in range(ch // 16):
                sv = src_v[r_, pl.ds(g * 16, 16)]
                dg = plsc.load_gather(dis_v, [sv])
                ew_v[r_, pl.ds(g * 16, 16)] = ew_v[r_, pl.ds(g * 16, 16)] * dg
                src_v[r_, pl.ds(g * 16, 16)] = sv * 2 + cid

        pl.loop(0, rpw)(remap)

        def fire_gather(rr, buf):
            @pl.when(rr < rpw)
            def _():
                pltpu.async_copy(xl_hbm.at[src_v.at[rr]], rows_v.at[buf], gsem)

        def wait_gather(rr, buf):
            pltpu.make_async_copy(
                xl_hbm.at[src_v.at[rr]], rows_v.at[buf], gsem
            ).wait()

        def fire_scatter(rr, buf):
            pltpu.async_copy(
                rows_v.at[buf], acc_sh.at[dst_v.at[rr]], ssem, add=True
            )

        def wait_scatter(rr, buf):
            pltpu.make_async_copy(
                rows_v.at[buf], acc_sh.at[dst_v.at[rr]], ssem
            ).wait()

        def scale(rr, buf):
            def sgroup(g):
                ewg = ew_v[rr, pl.ds(g * 16, 16)]
                for l in range(16):
                    sv = jnp.full((16,), ewg[l], jnp.float32)
                    e = g * 16 + l
                    for j in range(dh // 16):
                        rows_v[buf, e, pl.ds(j * 16, 16)] = (
                            rows_v[buf, e, pl.ds(j * 16, 16)] * sv
                        )

            pl.loop(0, ch // 16)(sgroup)

        fire_gather(0, 0)
        fire_gather(1, 1)
        wait_gather(0, 0)
        scale(0, 0)
        fire_scatter(0, 0)
        fire_gather(2, 2)

        def ring(r0):
            for q in range(3):
                rr = r0 + q
                buf = (1 + q) % 3
                wait_gather(rr, buf)
                scale(rr, buf)
                wait_scatter(rr - 1, (buf + 2) % 3)
                fire_scatter(rr, buf)
                fire_gather(rr + 2, (buf + 2) % 3)

        tail0 = 1 + 3 * ((rpw - 4) // 3)
        pl.loop(1, tail0, step=3)(ring)
        for rr in range(tail0, rpw):
            wait_gather(rr, rr % 3)
            scale(rr, rr % 3)
            wait_scatter(rr - 1, (rr - 1) % 3)
            fire_scatter(rr, rr % 3)
            fire_gather(rr + 2, (rr + 2) % 3)
        wait_scatter(rpw - 1, (rpw - 1) % 3)

        plsc.subcore_barrier()
        for t, (base, sizes) in enumerate(parts):
            @pl.when(sid == t)
            def _(base=base, sizes=sizes):
                off = 0
                for sz in sizes:
                    pltpu.sync_copy(
                        acc_sh.at[pl.ds(base + off, sz)],
                        stage_v.at[pl.ds(0, sz)],
                    )
                    pltpu.sync_copy(
                        stage_v.at[pl.ds(0, sz)],
                        msg_hbm.at[cid, pl.ds(base + off, sz)],
                    )
                    off += sz

    return kern(src3, dst3, ew3, xl2, degp)


def _tc_combine(degp_t, msg, x_lin, bias):
    n, d = x_lin.shape
    blk = 400

    def body(p_ref, m_ref, xl_ref, b_ref, o_ref):
        deg = 1.0 + p_ref[:, 0:1] + p_ref[:, 1:2]
        dis = lax.rsqrt(deg)
        m_full = jnp.concatenate([m_ref[0], m_ref[1]], axis=1)
        o_ref[...] = dis * m_full + (dis * dis) * xl_ref[...] + b_ref[...]

    return pl.pallas_call(
        body,
        grid=(n // blk,),
        in_specs=[
            pl.BlockSpec((blk, 2), lambda i: (i, 0)),
            pl.BlockSpec((2, blk, d // 2), lambda i: (0, i, 0)),
            pl.BlockSpec((blk, d), lambda i: (i, 0)),
            pl.BlockSpec((1, d), lambda i: (0, 0)),
        ],
        out_specs=pl.BlockSpec((blk, d), lambda i: (i, 0)),
        out_shape=jax.ShapeDtypeStruct((n, d), jnp.float32),
    )(degp_t, msg, x_lin, bias)


def kernel(x, edge_index, W_pred, b_pred, W_gcn, b_gcn):
    n, d = x.shape
    e = edge_index.shape[1]
    rpw = e // (_NW * _CH)
    rpw2 = e // (_NS * _CH)

    src = edge_index[0].astype(jnp.int32)
    dst = edge_index[1].astype(jnp.int32)
    src3 = src.reshape(_NW, rpw, _CH)
    dst3 = dst.reshape(_NW, rpw, _CH)

    w_ext = jnp.concatenate(
        [W_gcn, W_pred[:d], W_pred[d:], jnp.zeros((d, d - 2), jnp.float32)],
        axis=1,
    )
    bias_row = jnp.zeros((1, 2 * d), jnp.float32).at[0, d + 1].set(b_pred[0])

    xe = _tc_matmul(x, w_ext, bias_row)
    x_lin = xe[:, :d]
    a = xe[:, d]
    b = xe[:, d + 1]

    ew3, degp = _sc_edge_weights(src3, dst3, a, b)
    degp_t = degp.reshape(_NC, n).T

    msg, _ = _sc_scatter(
        src.reshape(_NS, rpw2, _CH),
        dst.reshape(_NS, rpw2, _CH),
        ew3.reshape(_NS, rpw2, _CH),
        x_lin.reshape(2 * n, d // 2),
        degp,
    )
    out = _tc_combine(degp_t, msg, x_lin, b_gcn.reshape(1, d))
    return out
